# baseline (speedup 1.0000x reference)
import jax
import jax.numpy as jnp
from jax.experimental import pallas as pl
from jax.experimental.pallas import tpu as pltpu

N, D = 8192, 1024
BM = 256


def _fused_body(x_ref, wdec_ref, bdec_ref, wl_ref, bl_ref, wr_ref, br_ref,
                out_ref, wl16_ref, wr16_ref):
    @pl.when(pl.program_id(0) == 0)
    def _cast_weights():
        wl16_ref[...] = wl_ref[...].astype(jnp.bfloat16)
        wr16_ref[...] = wr_ref[...].astype(jnp.bfloat16)

    xb = x_ref[...]
    dec = jnp.dot(xb, wdec_ref[...], preferred_element_type=jnp.float32)
    dec = dec + bdec_ref[0, 0]
    xb16 = xb.astype(jnp.bfloat16)
    left = jnp.dot(xb16, wl16_ref[...], preferred_element_type=jnp.float32)
    left = left + bl_ref[...]
    right = jnp.dot(xb16, wr16_ref[...], preferred_element_type=jnp.float32)
    right = right + br_ref[...]
    out_ref[...] = jnp.where(dec > 0.0, right, left)


def kernel(x, w_dec, b_dec, W_left, b_left, W_right, b_right):
    wdec2 = w_dec.reshape(D, 1)
    bdec2 = b_dec.reshape(1, 1)
    bl2 = b_left.reshape(1, D)
    br2 = b_right.reshape(1, D)
    return pl.pallas_call(
        _fused_body,
        grid=(N // BM,),
        in_specs=[
            pl.BlockSpec((BM, D), lambda i: (i, 0)),
            pl.BlockSpec((D, 1), lambda i: (0, 0)),
            pl.BlockSpec((1, 1), lambda i: (0, 0)),
            pl.BlockSpec((D, D), lambda i: (0, 0)),
            pl.BlockSpec((1, D), lambda i: (0, 0)),
            pl.BlockSpec((D, D), lambda i: (0, 0)),
            pl.BlockSpec((1, D), lambda i: (0, 0)),
        ],
        out_specs=pl.BlockSpec((BM, D), lambda i: (i, 0)),
        out_shape=jax.ShapeDtypeStruct((N, D), jnp.float32),
        scratch_shapes=[
            pltpu.VMEM((D, D), jnp.bfloat16),
            pltpu.VMEM((D, D), jnp.bfloat16),
        ],
    )(x, wdec2, bdec2, W_left, bl2, W_right, br2)


# speedup vs baseline: 1.1510x; 1.1510x over previous
import jax
import jax.numpy as jnp
from jax.experimental import pallas as pl
from jax.experimental.pallas import tpu as pltpu

N, D = 8192, 1024
BM = 1024


def _fused_body(x_ref, wdec_ref, bdec_ref, wl_ref, bl_ref, wr_ref, br_ref,
                out_ref, wl16_ref, wr16_ref):
    @pl.when(pl.program_id(0) == 0)
    def _cast_weights():
        wl16_ref[...] = wl_ref[...].astype(jnp.bfloat16)
        wr16_ref[...] = wr_ref[...].astype(jnp.bfloat16)

    xb = x_ref[...]
    dec = jnp.dot(xb, wdec_ref[...], preferred_element_type=jnp.float32)
    dec = dec + bdec_ref[0, 0]
    xb16 = xb.astype(jnp.bfloat16)
    left = jnp.dot(xb16, wl16_ref[...], preferred_element_type=jnp.float32)
    left = left + bl_ref[...]
    right = jnp.dot(xb16, wr16_ref[...], preferred_element_type=jnp.float32)
    right = right + br_ref[...]
    out_ref[...] = jnp.where(dec > 0.0, right, left)


def kernel(x, w_dec, b_dec, W_left, b_left, W_right, b_right):
    wdec2 = w_dec.reshape(D, 1)
    bdec2 = b_dec.reshape(1, 1)
    bl2 = b_left.reshape(1, D)
    br2 = b_right.reshape(1, D)
    return pl.pallas_call(
        _fused_body,
        grid=(N // BM,),
        in_specs=[
            pl.BlockSpec((BM, D), lambda i: (i, 0)),
            pl.BlockSpec((D, 1), lambda i: (0, 0)),
            pl.BlockSpec((1, 1), lambda i: (0, 0)),
            pl.BlockSpec((D, D), lambda i: (0, 0)),
            pl.BlockSpec((1, D), lambda i: (0, 0)),
            pl.BlockSpec((D, D), lambda i: (0, 0)),
            pl.BlockSpec((1, D), lambda i: (0, 0)),
        ],
        out_specs=pl.BlockSpec((BM, D), lambda i: (i, 0)),
        out_shape=jax.ShapeDtypeStruct((N, D), jnp.float32),
        scratch_shapes=[
            pltpu.VMEM((D, D), jnp.bfloat16),
            pltpu.VMEM((D, D), jnp.bfloat16),
        ],
    )(x, wdec2, bdec2, W_left, bl2, W_right, br2)
